# baseline (speedup 1.0000x reference)
import functools

import jax
import jax.numpy as jnp
import numpy as np
from jax import lax
from jax.experimental import pallas as pl
from jax.experimental.pallas import tpu as pltpu
from jax.experimental.pallas import tpu_sc as plsc

_SCALE = 3.0
_LANES = 16
_NC = 2
_NS = 16
_NW = _NC * _NS
_NB = 4


def _make_sc_lookup(B0, B1, V, D, dx):
    assert B0 % (128 * _NW) == 0 and D == 32 and B1 % 2 == 0
    CW = _NB * 128
    assert B0 == CW * _NW

    mesh = plsc.VectorSubcoreMesh(core_axis_name="c", subcore_axis_name="s")
    mul = float(np.float32(dx) * (np.float32(1.0) / np.float32(_SCALE)))
    add = float(dx)
    hi = float(V - 1)

    @functools.partial(
        pl.kernel,
        mesh=mesh,
        out_type=jax.ShapeDtypeStruct((B1, (D // 8) * B0 * 8), jnp.float32),
        scratch_types=[
            pltpu.VMEM((2, CW), jnp.float32),
            pltpu.VMEM((2, CW), jnp.int32),
            pltpu.VMEM((2, CW, D), jnp.float32),
            pltpu.VMEM((2, (D // 8) * _NB * 8 * 128), jnp.float32),
            pltpu.SemaphoreType.DMA,
            pltpu.SemaphoreType.DMA,
            pltpu.SemaphoreType.DMA,
            pltpu.SemaphoreType.DMA,
            pltpu.SemaphoreType.DMA,
            pltpu.SemaphoreType.DMA,
        ],
        compiler_params=pltpu.CompilerParams(use_tc_tiling_on_sc=False,
                                             needs_layout_passes=False),
    )
    def lookup(post_hbm, tab_hbm, out_hbm, pos_v, idx_v, rows_v, tr_v,
               sp0, sp1, sg0, sg1, sw0, sw1):
        wid = lax.axis_index("s") * _NC + lax.axis_index("c")
        base = wid * CW
        sp, sg, sw = (sp0, sp1), (sg0, sg1), (sw0, sw1)
        iota16 = lax.iota(jnp.int32, _LANES)

        def pos_cp(n, b):
            return pltpu.make_async_copy(
                post_hbm.at[n, pl.ds(base, CW)], pos_v.at[b], sp[b])

        def gat_cp(b, k):
            return pltpu.make_async_copy(
                tab_hbm.at[idx_v.at[b, pl.ds(k * 128, 128)]],
                rows_v.at[b, pl.ds(k * 128, 128)],
                sg[b])

        def out_cps(n, b):
            seg = _NB * 8 * 128
            return [
                pltpu.make_async_copy(
                    tr_v.at[b, pl.ds(db * seg, seg)],
                    out_hbm.at[n, pl.ds(db * (B0 * 8) + wid * seg, seg)],
                    sw[b])
                for db in range(D // 8)
            ]

        def quantize(b):
            @plsc.parallel_loop(0, CW // _LANES, unroll=4)
            def _(i):
                p = pos_v[b, pl.ds(i * _LANES, _LANES)]
                t = p * mul
                t = t + add
                t = jnp.minimum(jnp.maximum(t, 0.0), hi)
                idx_v[b, pl.ds(i * _LANES, _LANES)] = t.astype(jnp.int32)

        seg = _NB * 8 * 128
        offv = []
        for h in range(2):
            dd = h * _LANES + iota16
            offv.append((dd >> 3) * seg + (dd & 7) * 128)

        def transpose(b):
            @plsc.parallel_loop(0, CW)
            def _(r):
                sb = (r >> 7) * 1024 + (r & 127)
                for h in range(2):
                    v = rows_v[b, r, pl.ds(h * _LANES, _LANES)]
                    plsc.store_scatter(tr_v.at[b], [offv[h] + sb], v)

        pos_cp(0, 0).start()
        pos_cp(1, 1).start()

        def outer(g, carry):
            for b in range(2):
                n = g * 2 + b
                pb = 1 - b
                pos_cp(n, b).wait()
                quantize(b)

                @pl.when(n + 2 < B1)
                def _():
                    pos_cp(n + 2, b).start()

                for k in range(_NB):
                    gat_cp(b, k).start()

                @pl.when(n >= 1)
                def _():
                    for k in range(_NB):
                        gat_cp(pb, k).wait()

                    @pl.when(n >= 3)
                    def _():
                        for cp in out_cps(n - 3, pb):
                            cp.wait()

                    transpose(pb)
                    for cp in out_cps(n - 1, pb):
                        cp.start()
            return carry

        lax.fori_loop(0, B1 // 2, outer, 0)

        for k in range(_NB):
            gat_cp(1, k).wait()
        for cp in out_cps(B1 - 3, 1):
            cp.wait()
        transpose(1)
        for cp in out_cps(B1 - 1, 1):
            cp.start()
        for cp in out_cps(B1 - 2, 0):
            cp.wait()
        for cp in out_cps(B1 - 1, 1):
            cp.wait()

    return lookup


def kernel(pos, embedding):
    B0, B1 = pos.shape
    V, D = embedding.shape
    dx = (V - 1) // 2
    post = jnp.swapaxes(pos, 0, 1)
    out2 = _make_sc_lookup(B0, B1, V, D, dx)(post, embedding)
    out5 = out2.reshape(B1, D // 8, B0 // 128, 8, 128)
    return out5.transpose(2, 4, 0, 1, 3).reshape(B0, B1, D)


# speedup vs baseline: 2.2662x; 2.2662x over previous
import functools

import jax
import jax.numpy as jnp
import numpy as np
from jax import lax
from jax.experimental import pallas as pl
from jax.experimental.pallas import tpu as pltpu
from jax.experimental.pallas import tpu_sc as plsc

_SCALE = 3.0
_LANES = 16
_NC = 2
_NS = 16
_NW = _NC * _NS
_NB = 4


def _make_sc_lookup(B0, B1, V, D, dx):
    assert B0 % (128 * _NW) == 0 and D == 32 and B1 % 2 == 0
    CW = _NB * 128
    assert B0 == CW * _NW

    mesh = plsc.VectorSubcoreMesh(core_axis_name="c", subcore_axis_name="s")
    mul = float(np.float32(dx) * (np.float32(1.0) / np.float32(_SCALE)))
    add = float(dx)
    hi = float(V - 1)

    @functools.partial(
        pl.kernel,
        mesh=mesh,
        out_type=jax.ShapeDtypeStruct((B1, (D // 8) * B0 * 8), jnp.float32),
        scratch_types=[
            pltpu.VMEM((2, CW), jnp.float32),
            pltpu.VMEM((2, CW), jnp.int32),
            pltpu.VMEM((2, CW, D), jnp.float32),
            pltpu.VMEM((2, (D // 8) * _NB * 8 * 128), jnp.float32),
            pltpu.SemaphoreType.DMA,
            pltpu.SemaphoreType.DMA,
            pltpu.SemaphoreType.DMA,
            pltpu.SemaphoreType.DMA,
            pltpu.SemaphoreType.DMA,
            pltpu.SemaphoreType.DMA,
        ],
        compiler_params=pltpu.CompilerParams(use_tc_tiling_on_sc=False,
                                             needs_layout_passes=False),
    )
    def lookup(post_hbm, tab_hbm, out_hbm, pos_v, idx_v, rows_v, tr_v,
               sp0, sp1, sg0, sg1, sw0, sw1):
        wid = lax.axis_index("s") * _NC + lax.axis_index("c")
        base = wid * CW
        sp, sg, sw = (sp0, sp1), (sg0, sg1), (sw0, sw1)
        iota16 = lax.iota(jnp.int32, _LANES)

        def pos_cp(n, b):
            return pltpu.make_async_copy(
                post_hbm.at[n, pl.ds(base, CW)], pos_v.at[b], sp[b])

        def gat_cp(b, k):
            return pltpu.make_async_copy(
                tab_hbm.at[idx_v.at[b, pl.ds(k * 128, 128)]],
                rows_v.at[b, pl.ds(k * 128, 128)],
                sg[b])

        def out_cps(n, b):
            seg = _NB * 8 * 128
            return [
                pltpu.make_async_copy(
                    tr_v.at[b, pl.ds(db * seg, seg)],
                    out_hbm.at[n, pl.ds(db * (B0 * 8) + wid * seg, seg)],
                    sw[b])
                for db in range(D // 8)
            ]

        def quantize(b):
            @plsc.parallel_loop(0, CW // _LANES, unroll=4)
            def _(i):
                p = pos_v[b, pl.ds(i * _LANES, _LANES)]
                t = p * mul
                t = t + add
                t = jnp.minimum(jnp.maximum(t, 0.0), hi)
                idx_v[b, pl.ds(i * _LANES, _LANES)] = t.astype(jnp.int32)

        seg = _NB * 8 * 128

        def transpose(b):
            @plsc.parallel_loop(0, D)
            def _(d):
                cidx = jnp.bitwise_xor(d, iota16)
                ovi = (cidx >> 3) * seg + (cidx & 7) * 128 + iota16
                for g in range(CW // _LANES):
                    ridx = iota16 + g * _LANES
                    v = plsc.load_gather(rows_v.at[b], [ridx, cidx])
                    offs = ovi + ((g // 8) * 1024 + (g * _LANES) % 128)
                    plsc.store_scatter(tr_v.at[b], [offs], v)

        pos_cp(0, 0).start()
        pos_cp(1, 1).start()

        def outer(g, carry):
            for b in range(2):
                n = g * 2 + b
                pb = 1 - b
                pos_cp(n, b).wait()
                quantize(b)

                @pl.when(n + 2 < B1)
                def _():
                    pos_cp(n + 2, b).start()

                for k in range(_NB):
                    gat_cp(b, k).start()

                @pl.when(n >= 1)
                def _():
                    for k in range(_NB):
                        gat_cp(pb, k).wait()

                    @pl.when(n >= 3)
                    def _():
                        for cp in out_cps(n - 3, pb):
                            cp.wait()

                    transpose(pb)
                    for cp in out_cps(n - 1, pb):
                        cp.start()
            return carry

        lax.fori_loop(0, B1 // 2, outer, 0)

        for k in range(_NB):
            gat_cp(1, k).wait()
        for cp in out_cps(B1 - 3, 1):
            cp.wait()
        transpose(1)
        for cp in out_cps(B1 - 1, 1):
            cp.start()
        for cp in out_cps(B1 - 2, 0):
            cp.wait()
        for cp in out_cps(B1 - 1, 1):
            cp.wait()

    return lookup


def kernel(pos, embedding):
    B0, B1 = pos.shape
    V, D = embedding.shape
    dx = (V - 1) // 2
    post = jnp.swapaxes(pos, 0, 1)
    out2 = _make_sc_lookup(B0, B1, V, D, dx)(post, embedding)
    out5 = out2.reshape(B1, D // 8, B0 // 128, 8, 128)
    return out5.transpose(2, 4, 0, 1, 3).reshape(B0, B1, D)
